# Initial kernel scaffold; baseline (speedup 1.0000x reference)
import jax
import jax.numpy as jnp
from jax.experimental import pallas as pl


def kernel(raw_features, feature_means, feature_stds, bin_boundaries, emb_tables):
    raise NotImplementedError("write your pallas kernel here")


# speedup vs baseline: 8.4212x; 8.4212x over previous
import jax
import jax.numpy as jnp
from jax import lax
from jax.experimental import pallas as pl
from jax.experimental.pallas import tpu as pltpu
from jax.experimental.pallas import tpu_sc as plsc

NUM_FEATURES = 7
NUM_BINS = 10
EMB_DIM = 18
BATCH = 16384
NUM_INNER = NUM_BINS - 1

NC = 2
NS = 16
NW = NC * NS
LANES = 16

ROWS_PER_W = BATCH // NW
FLAT_PER_W = ROWS_PER_W * NUM_FEATURES
OUT_PER_W = FLAT_PER_W * EMB_DIM
GROUPS = ROWS_PER_W // LANES
TABLE_SIZE = NUM_FEATURES * NUM_BINS * EMB_DIM
BOUNDS_SIZE = NUM_FEATURES * NUM_INNER * LANES
MS_SIZE = NUM_FEATURES * LANES


def _sc_body(raw_hbm, bounds_hbm, means_hbm, stds_hbm, table_hbm, out_hbm,
             raw_v, bounds_v, means_v, stds_v, table_v, rows_v):
    wid = lax.axis_index("s") * NC + lax.axis_index("c")
    base = wid * FLAT_PER_W

    pltpu.sync_copy(raw_hbm.at[pl.ds(base, FLAT_PER_W)], raw_v)
    pltpu.sync_copy(bounds_hbm, bounds_v)
    pltpu.sync_copy(means_hbm, means_v)
    pltpu.sync_copy(stds_hbm, stds_v)
    pltpu.sync_copy(table_hbm, table_v)

    lane = lax.broadcasted_iota(jnp.int32, (LANES,), 0)
    lane_f = lane * NUM_FEATURES
    lane_fe = lane_f * EMB_DIM

    for i in range(NUM_FEATURES):
        mean_i = means_v[pl.ds(i * LANES, LANES)]
        std_i = stds_v[pl.ds(i * LANES, LANES)]
        bvecs = [bounds_v[pl.ds((i * NUM_INNER + k) * LANES, LANES)]
                 for k in range(NUM_INNER)]

        def body(g, carry, i=i, mean_i=mean_i, std_i=std_i, bvecs=bvecs):
            flat0 = g * (LANES * NUM_FEATURES) + i
            gidx = lane_f + flat0
            vals = plsc.load_gather(raw_v, [gidx])
            x = (vals - mean_i) / (std_i + 1e-8)
            cnt = jnp.zeros((LANES,), jnp.int32)
            for k in range(NUM_INNER):
                cnt = cnt + (x > bvecs[k]).astype(jnp.int32)
            src0 = (cnt + i * NUM_BINS) * EMB_DIM
            dst0 = lane_fe + flat0 * EMB_DIM
            for d in range(EMB_DIM):
                e = plsc.load_gather(table_v, [src0 + d])
                plsc.store_scatter(rows_v, [dst0 + d], e)
            return carry

        lax.fori_loop(0, GROUPS, body, 0)

    pltpu.sync_copy(rows_v, out_hbm.at[pl.ds(wid * OUT_PER_W, OUT_PER_W)])


@jax.jit
def _encode(raw_flat, bounds_b, means_b, stds_b, table_flat):
    mesh = plsc.VectorSubcoreMesh(
        core_axis_name="c", subcore_axis_name="s",
        num_cores=NC, num_subcores=NS,
    )
    return pl.kernel(
        _sc_body,
        out_type=jax.ShapeDtypeStruct((BATCH * NUM_FEATURES * EMB_DIM,),
                                      jnp.float32),
        mesh=mesh,
        compiler_params=pltpu.CompilerParams(needs_layout_passes=False),
        scratch_types=[
            pltpu.VMEM((FLAT_PER_W,), jnp.float32),
            pltpu.VMEM((BOUNDS_SIZE,), jnp.float32),
            pltpu.VMEM((MS_SIZE,), jnp.float32),
            pltpu.VMEM((MS_SIZE,), jnp.float32),
            pltpu.VMEM((TABLE_SIZE,), jnp.float32),
            pltpu.VMEM((OUT_PER_W,), jnp.float32),
        ],
    )(raw_flat, bounds_b, means_b, stds_b, table_flat)


def kernel(raw_features, feature_means, feature_stds, bin_boundaries,
           emb_tables):
    raw_flat = raw_features.reshape(BATCH * NUM_FEATURES)
    inner = bin_boundaries[:, 1:-1]
    bounds_b = jnp.broadcast_to(
        inner[:, :, None], (NUM_FEATURES, NUM_INNER, LANES)).reshape(-1)
    means_b = jnp.broadcast_to(
        feature_means[:, None], (NUM_FEATURES, LANES)).reshape(-1)
    stds_b = jnp.broadcast_to(
        feature_stds[:, None], (NUM_FEATURES, LANES)).reshape(-1)
    table_flat = emb_tables.reshape(TABLE_SIZE)
    out = _encode(raw_flat, bounds_b, means_b, stds_b, table_flat)
    return out.reshape(BATCH, NUM_FEATURES * EMB_DIM)
